# Initial kernel scaffold; baseline (speedup 1.0000x reference)
import jax
import jax.numpy as jnp
from jax.experimental import pallas as pl


def kernel(users, items, edge_index, edge_vals, user_emb, item_emb):
    raise NotImplementedError("write your pallas kernel here")


# speedup vs baseline: 2.8845x; 2.8845x over previous
import functools

import jax
import jax.numpy as jnp
from jax import lax
from jax.experimental import pallas as pl
from jax.experimental.pallas import tpu as pltpu
from jax.experimental.pallas import tpu_sc as plsc

N_USERS = 25000
N_ITEMS = 25000
N_NODES = N_USERS + N_ITEMS
E = 800000
D = 64
NL = 3
B = 4096

NC = 2
NS = 16
L = 16

HALF = N_NODES // NC
ACC_ROWS = 25600
K = 400
KS = 80
NSUB = K // KS
CHUNKS = E // (NS * K)
WB = 100
WB_CHUNKS = HALF // WB

_mesh = plsc.VectorSubcoreMesh(core_axis_name="c", subcore_axis_name="s")
_params = pltpu.CompilerParams(use_tc_tiling_on_sc=False,
                               needs_layout_passes=False)


def _layer_body(table, srcr, dstr, wr, sum_in, table_out, sum_out,
                src_i, dst_i, dloc, w_v, rows, accbuf, sumbuf, acc,
                sem0, sem1):
    c = lax.axis_index("c")
    s = lax.axis_index("s")
    base_node = c * HALF
    sems = (sem0, sem1)

    def _zrow(r, _):
        for j in range(D // L):
            accbuf[r, pl.ds(j * L, L)] = jnp.zeros((L,), jnp.float32)
        return 0
    lax.fori_loop(0, WB, _zrow, 0)
    for q in range(1600 // WB):
        pltpu.sync_copy(accbuf, acc.at[pl.ds(s * 1600 + q * WB, WB)])
    plsc.subcore_barrier()

    def _chunk(ci, _):
        row = s * CHUNKS + ci
        pltpu.sync_copy(srcr.at[row], src_i)
        pltpu.sync_copy(dstr.at[row], dst_i)
        pltpu.sync_copy(wr.at[row], w_v)
        for j in range(NSUB):
            for g in range(KS // L):
                v = dst_i[j, pl.ds(g * L, L)] - base_node
                ok = (v >= 0) & (v < HALF)
                dloc[j, pl.ds(g * L, L)] = jnp.where(ok, v, HALF)
        gathers = [None] * NSUB
        gathers[0] = pltpu.async_copy(table.at[src_i.at[0]], rows.at[0],
                                      sems[0])
        for j in range(NSUB):
            b = j % 2
            if j + 1 < NSUB:
                gathers[j + 1] = pltpu.async_copy(
                    table.at[src_i.at[j + 1]], rows.at[(j + 1) % 2],
                    sems[(j + 1) % 2])
            gathers[j].wait()

            def _mul(g, _):
                w16 = w_v[pl.ds(j * KS + g * L, L)]
                for t in range(L):
                    e = g * L + t
                    we = w16[t]
                    for q in range(D // L):
                        sl = pl.ds(q * L, L)
                        rows[b, e, sl] = rows[b, e, sl] * we
                return 0
            lax.fori_loop(0, KS // L, _mul, 0)
            pltpu.sync_copy(rows.at[b], acc.at[dloc.at[j]], add=True)
        return 0

    lax.fori_loop(0, CHUNKS, _chunk, 0)
    plsc.subcore_barrier()

    for i in range((WB_CHUNKS + NS - 1) // NS):
        wc = s + NS * i

        @pl.when(wc < WB_CHUNKS)
        def _():
            gbase = base_node + wc * WB
            pltpu.sync_copy(acc.at[pl.ds(wc * WB, WB)], accbuf)
            pltpu.sync_copy(sum_in.at[pl.ds(gbase, WB)], sumbuf)

            def _add(r, _):
                for j in range(D // L):
                    sl = pl.ds(j * L, L)
                    sumbuf[r, sl] = sumbuf[r, sl] + accbuf[r, sl]
                return 0
            lax.fori_loop(0, WB, _add, 0)
            pltpu.sync_copy(accbuf, table_out.at[pl.ds(gbase, WB)])
            pltpu.sync_copy(sumbuf, sum_out.at[pl.ds(gbase, WB)])


_layer_call = functools.partial(
    pl.kernel,
    out_type=(jax.ShapeDtypeStruct((N_NODES, D), jnp.float32),
              jax.ShapeDtypeStruct((N_NODES, D), jnp.float32)),
    mesh=_mesh,
    compiler_params=_params,
    scratch_types=[
        pltpu.VMEM((NSUB, KS), jnp.int32),
        pltpu.VMEM((NSUB, KS), jnp.int32),
        pltpu.VMEM((NSUB, KS), jnp.int32),
        pltpu.VMEM((K,), jnp.float32),
        pltpu.VMEM((2, KS, D), jnp.float32),
        pltpu.VMEM((WB, D), jnp.float32),
        pltpu.VMEM((WB, D), jnp.float32),
        pltpu.VMEM_SHARED((ACC_ROWS, D), jnp.float32),
        pltpu.SemaphoreType.DMA,
        pltpu.SemaphoreType.DMA,
    ],
)(_layer_body)


BPW = B // (NC * NS)


def _final_body(sum_t, users, items, gamma, uidx, iidx, rows_u, rows_i,
                gout, sem):
    c = lax.axis_index("c")
    s = lax.axis_index("s")
    wid = s * NC + c
    base = wid * BPW
    pltpu.sync_copy(users.at[pl.ds(base, BPW)], uidx)
    pltpu.sync_copy(items.at[pl.ds(base, BPW)], iidx)
    for g in range(BPW // L):
        sl = pl.ds(g * L, L)
        iidx[sl] = iidx[sl] + N_USERS
    pltpu.async_copy(sum_t.at[uidx], rows_u, sem).wait()
    pltpu.async_copy(sum_t.at[iidx], rows_i, sem).wait()

    def _dot(g, _):
        rowv = g * L + lax.iota(jnp.int32, L)
        acc16 = jnp.zeros((L,), jnp.float32)

        def _dim(d, a):
            colv = jnp.full((L,), d, jnp.int32)
            cu = plsc.load_gather(rows_u, [rowv, colv])
            ci = plsc.load_gather(rows_i, [rowv, colv])
            return a + cu * ci
        acc16 = lax.fori_loop(0, D, _dim, acc16)
        gout[pl.ds(g * L, L)] = acc16 * jnp.float32(1.0 / ((NL + 1) * (NL + 1)))
        return 0
    lax.fori_loop(0, BPW // L, _dot, 0)
    pltpu.sync_copy(gout, gamma.at[pl.ds(base, BPW)])


_final_call = functools.partial(
    pl.kernel,
    out_type=jax.ShapeDtypeStruct((B,), jnp.float32),
    mesh=_mesh,
    compiler_params=_params,
    scratch_types=[
        pltpu.VMEM((BPW,), jnp.int32),
        pltpu.VMEM((BPW,), jnp.int32),
        pltpu.VMEM((BPW, D), jnp.float32),
        pltpu.VMEM((BPW, D), jnp.float32),
        pltpu.VMEM((BPW,), jnp.float32),
        pltpu.SemaphoreType.DMA,
    ],
)(_final_body)


def kernel(users, items, edge_index, edge_vals, user_emb, item_emb):
    src = edge_index[0].astype(jnp.int32).reshape(NS * CHUNKS, NSUB, KS)
    dst = edge_index[1].astype(jnp.int32).reshape(NS * CHUNKS, NSUB, KS)
    w = edge_vals.astype(jnp.float32).reshape(NS * CHUNKS, K)
    table = jnp.concatenate([user_emb, item_emb], axis=0)
    sum_t = table
    for _ in range(NL):
        table, sum_t = _layer_call(table, src, dst, w, sum_t)
    return _final_call(sum_t, users.astype(jnp.int32), items.astype(jnp.int32))


# speedup vs baseline: 6.6258x; 2.2970x over previous
import functools

import jax
import jax.numpy as jnp
from jax import lax
from jax.experimental import pallas as pl
from jax.experimental.pallas import tpu as pltpu
from jax.experimental.pallas import tpu_sc as plsc

N_USERS = 25000
N_ITEMS = 25000
N_NODES = N_USERS + N_ITEMS
E = 800000
D = 64
NL = 3
B = 4096

NC = 2
NS = 16
L = 16

HALF = N_NODES // NC
ACC_ROWS = 25600
K = 400
KS = 80
NSUB = K // KS
CHUNKS = E // (NS * K)
WBR = 1000
WB_CHUNKS = HALF // WBR

_mesh = plsc.VectorSubcoreMesh(core_axis_name="c", subcore_axis_name="s")
_params = pltpu.CompilerParams(use_tc_tiling_on_sc=False,
                               needs_layout_passes=False)


def _layer_body(table, srcr, dstr, wr, table_out,
                src_i, dl, w_v, gbuf, sbuf, acc, g0, g1, s0, s1, si):
    c = lax.axis_index("c")
    s = lax.axis_index("s")
    base_node = c * HALF
    gsem = (g0, g1)
    ssem = (s0, s1)

    def _zrow(r, _):
        for j in range(D // L):
            sbuf[0, r, pl.ds(j * L, L)] = jnp.zeros((L,), jnp.float32)
        return 0
    lax.fori_loop(0, KS, _zrow, 0)
    zd = [pltpu.async_copy(sbuf.at[0], acc.at[pl.ds(s * 1600 + q * KS, KS)],
                           s0) for q in range(1600 // KS)]
    for d in zd:
        d.wait()
    plsc.subcore_barrier()

    def _drain_scatter(b):
        pltpu.make_async_copy(sbuf.at[b], acc.at[dl.at[0, 0]], ssem[b]).wait()

    def _do_chunk(ci, p, first, fire_next):
        row = s * CHUNKS + ci
        pltpu.make_async_copy(srcr.at[row], src_i.at[p], si).wait()
        pltpu.make_async_copy(dstr.at[row], dl.at[p], si).wait()
        pltpu.make_async_copy(wr.at[row], w_v.at[p], si).wait()
        if fire_next:
            nrow = row + 1
            np_ = 1 - p
            pltpu.async_copy(srcr.at[nrow], src_i.at[np_], si)
            pltpu.async_copy(dstr.at[nrow], dl.at[np_], si)
            pltpu.async_copy(wr.at[nrow], w_v.at[np_], si)
        for j in range(NSUB):
            for g in range(KS // L):
                v = dl[p, j, pl.ds(g * L, L)] - base_node
                ok = (v >= 0) & (v < HALF)
                dl[p, j, pl.ds(g * L, L)] = jnp.where(ok, v, HALF)
        gat = [None] * NSUB
        for j in range(2):
            gat[j] = pltpu.async_copy(table.at[src_i.at[p, j]],
                                      gbuf.at[j % 2], gsem[j % 2])
        last = [None, None]
        for j in range(NSUB):
            b = j % 2
            gat[j].wait()
            if j >= 2:
                last[b].wait()
            elif not first:
                _drain_scatter(b)

            def _mul(g, _):
                w16 = w_v[p, pl.ds(j * KS + g * L, L)]
                for t in range(L):
                    e = g * L + t
                    we = w16[t]
                    for q in range(D // L):
                        sl = pl.ds(q * L, L)
                        sbuf[b, e, sl] = gbuf[b, e, sl] * we
                return 0
            lax.fori_loop(0, KS // L, _mul, 0)
            last[b] = pltpu.async_copy(sbuf.at[b], acc.at[dl.at[p, j]],
                                       ssem[b], add=True)
            if j + 2 < NSUB:
                gat[j + 2] = pltpu.async_copy(table.at[src_i.at[p, j + 2]],
                                              gbuf.at[b], gsem[b])
        return last

    row0 = s * CHUNKS
    pltpu.async_copy(srcr.at[row0], src_i.at[0], si)
    pltpu.async_copy(dstr.at[row0], dl.at[0], si)
    pltpu.async_copy(wr.at[row0], w_v.at[0], si)

    def _pair(i, _):
        @pl.when(i == 0)
        def _():
            _do_chunk(2 * i, 0, True, True)

        @pl.when(i > 0)
        def _():
            _do_chunk(2 * i, 0, False, True)
        _do_chunk(2 * i + 1, 1, False, True)
        return 0

    lax.fori_loop(0, (CHUNKS - 1) // 2, _pair, 0)
    last = _do_chunk(CHUNKS - 1, 0, False, False)
    last[1].wait()
    last[0].wait()
    plsc.subcore_barrier()

    for i in range((WB_CHUNKS + NS - 1) // NS):
        wc = s + NS * i

        @pl.when(wc < WB_CHUNKS)
        def _():
            pltpu.sync_copy(acc.at[pl.ds(wc * WBR, WBR)],
                            table_out.at[pl.ds(base_node + wc * WBR, WBR)])


_layer_call = functools.partial(
    pl.kernel,
    out_type=jax.ShapeDtypeStruct((N_NODES, D), jnp.float32),
    mesh=_mesh,
    compiler_params=_params,
    scratch_types=[
        pltpu.VMEM((2, NSUB, KS), jnp.int32),
        pltpu.VMEM((2, NSUB, KS), jnp.int32),
        pltpu.VMEM((2, K), jnp.float32),
        pltpu.VMEM((2, KS, D), jnp.float32),
        pltpu.VMEM((2, KS, D), jnp.float32),
        pltpu.VMEM_SHARED((ACC_ROWS, D), jnp.float32),
        pltpu.SemaphoreType.DMA,
        pltpu.SemaphoreType.DMA,
        pltpu.SemaphoreType.DMA,
        pltpu.SemaphoreType.DMA,
        pltpu.SemaphoreType.DMA,
    ],
)(_layer_body)


BPW = B // (NC * NS)


def _final_body(t0, t1, t2, t3, users, items, gamma,
                uidx, iidx, rbuf, usum, isum, gout, sem):
    c = lax.axis_index("c")
    s = lax.axis_index("s")
    wid = s * NC + c
    base = wid * BPW
    pltpu.sync_copy(users.at[pl.ds(base, BPW)], uidx)
    pltpu.sync_copy(items.at[pl.ds(base, BPW)], iidx)
    for g in range(BPW // L):
        sl = pl.ds(g * L, L)
        iidx[sl] = iidx[sl] + N_USERS

    def _acc_rows(idx, dst):
        pltpu.async_copy(t0.at[idx], dst, sem).wait()
        for t in (t1, t2, t3):
            pltpu.async_copy(t.at[idx], rbuf, sem).wait()

            def _add(r, _):
                for j in range(D // L):
                    sl = pl.ds(j * L, L)
                    dst[r, sl] = dst[r, sl] + rbuf[r, sl]
                return 0
            lax.fori_loop(0, BPW, _add, 0)

    _acc_rows(uidx, usum)
    _acc_rows(iidx, isum)

    def _dot(g, _):
        rowv = g * L + lax.iota(jnp.int32, L)
        acc16 = jnp.zeros((L,), jnp.float32)

        def _dim(d, a):
            colv = jnp.full((L,), d, jnp.int32)
            cu = plsc.load_gather(usum, [rowv, colv])
            ci = plsc.load_gather(isum, [rowv, colv])
            return a + cu * ci
        acc16 = lax.fori_loop(0, D, _dim, acc16)
        gout[pl.ds(g * L, L)] = acc16 * jnp.float32(1.0 / ((NL + 1) * (NL + 1)))
        return 0
    lax.fori_loop(0, BPW // L, _dot, 0)
    pltpu.sync_copy(gout, gamma.at[pl.ds(base, BPW)])


_final_call = functools.partial(
    pl.kernel,
    out_type=jax.ShapeDtypeStruct((B,), jnp.float32),
    mesh=_mesh,
    compiler_params=_params,
    scratch_types=[
        pltpu.VMEM((BPW,), jnp.int32),
        pltpu.VMEM((BPW,), jnp.int32),
        pltpu.VMEM((BPW, D), jnp.float32),
        pltpu.VMEM((BPW, D), jnp.float32),
        pltpu.VMEM((BPW, D), jnp.float32),
        pltpu.VMEM((BPW,), jnp.float32),
        pltpu.SemaphoreType.DMA,
    ],
)(_final_body)


def kernel(users, items, edge_index, edge_vals, user_emb, item_emb):
    src = edge_index[0].astype(jnp.int32).reshape(NS * CHUNKS, NSUB, KS)
    dst = edge_index[1].astype(jnp.int32).reshape(NS * CHUNKS, NSUB, KS)
    w = edge_vals.astype(jnp.float32).reshape(NS * CHUNKS, K)
    t0 = jnp.concatenate([user_emb, item_emb], axis=0)
    t1 = _layer_call(t0, src, dst, w)
    t2 = _layer_call(t1, src, dst, w)
    t3 = _layer_call(t2, src, dst, w)
    return _final_call(t0, t1, t2, t3,
                       users.astype(jnp.int32), items.astype(jnp.int32))
